# baseline (speedup 1.0000x reference)
import jax
import jax.numpy as jnp
import numpy as np
from jax.experimental import pallas as pl
from jax.experimental.pallas import tpu as pltpu

T = 2048
D = 768
E = 8
F = 4 * D
C = int(np.ceil(T / E * 1.25))
KKEEP = max(1, int(round(T * (1.0 - 0.6))))
V_MAX = 3.0
FEAT_CLAMP = 10.0
_L = 16
_NCH = T // _L
_NF = 3
_FB = F // _NF

_NT = (((1,), (1,)), ((), ()))
_TN = (((0,), (0,)), ((), ()))


def _moe_kernel(x_ref, WcatT_ref, bcat_ref, W1_ref, b1_ref, W2_ref, b2_ref,
                Wv_ref, bv_ref, Wo_ref, bo_ref, bk_ref,
                out_ref, maskT_ref, sp_ref, nc_ref,
                xbf, eS, pS, gcol, PS, bufS, ybuf, acc):
    e = pl.program_id(0)
    f = pl.program_id(1)

    @pl.when((e == 0) & (f == 0))
    def _route():
        xb = x_ref[...].astype(jnp.bfloat16)
        xbf[...] = xb
        wct = WcatT_ref[...]
        lgT = jax.lax.dot_general(
            wct, xb, _NT, preferred_element_type=jnp.float32) + bcat_ref[...]
        lg = lgT[:E, :]
        mx = jnp.max(lg, axis=0, keepdims=True)
        subi = jax.lax.broadcasted_iota(jnp.int32, (E, T), 0)
        eT = jnp.min(jnp.where(lg == mx, subi, E), axis=0, keepdims=True)
        eS[...] = eT

        onehot = (subi == eT).astype(jnp.float32)
        lane = jax.lax.broadcasted_iota(jnp.int32, (E, T), 1)
        cs = onehot
        sft = 1
        while sft < T:
            r = jnp.roll(cs, sft, axis=1)
            cs = cs + jnp.where(lane >= sft, r, 0.0)
            sft *= 2
        posT = jnp.sum(cs * onehot, axis=0, keepdims=True) - 1.0
        pS[...] = posT.astype(jnp.int32)

        lgC = jax.lax.dot_general(
            xb, wct, _NT, preferred_element_type=jnp.float32) \
            + bcat_ref[...].reshape(1, 16)
        r8 = lgC[:, :E]
        mxc = jnp.max(r8, axis=1, keepdims=True)
        sc_ = jnp.sum(jnp.exp(r8 - mxc), axis=1, keepdims=True)
        gcol[...] = 1.0 / sc_

        sc = lgT[E:E + 1, :]
        bi = jax.lax.bitcast_convert_type(sc, jnp.int32)
        key = jnp.where(bi < 0, bi ^ jnp.int32(0x7FFFFFFF), bi)

        def count_ge(m):
            return jnp.sum((key >= m).astype(jnp.float32))

        kf = jnp.float32(KKEEP)
        m0 = jnp.where(count_ge(jnp.int32(0)) >= kf,
                       jnp.int32(0), jnp.int32(-2147483648))

        def body(i, m):
            mt = m | jax.lax.shift_left(jnp.int32(1), 30 - i)
            return jnp.where(count_ge(mt) >= kf, mt, m)

        m = jax.lax.fori_loop(0, 31, body, m0)
        maskT = (key >= m).astype(jnp.float32)
        maskT_ref[...] = maskT
        nc = jnp.sum(maskT, axis=1, keepdims=True)
        nc_ref[...] = nc
        sp_ref[...] = 1.0 - nc / jnp.float32(T)

    @pl.when(f == 0)
    def _dispatch():
        iota0 = jax.lax.broadcasted_iota(jnp.int32, (C, T), 0)
        pm = jnp.where(eS[...] == e, pS[...], -1)
        P = (iota0 == pm).astype(jnp.bfloat16)
        PS[...] = P
        bufS[...] = jax.lax.dot(
            P, xbf[...], preferred_element_type=jnp.float32
        ).astype(jnp.bfloat16)

    w1 = W1_ref[0].astype(jnp.bfloat16)
    h = jax.nn.relu(jax.lax.dot(bufS[...], w1,
                                preferred_element_type=jnp.float32)
                    + b1_ref[0])
    w2 = W2_ref[0].astype(jnp.bfloat16)
    part = jax.lax.dot(h.astype(jnp.bfloat16), w2,
                       preferred_element_type=jnp.float32)

    @pl.when(f == 0)
    def _():
        ybuf[...] = part

    @pl.when(f != 0)
    def _():
        ybuf[...] = ybuf[...] + part

    @pl.when(f == _NF - 1)
    def _combine():
        y = ybuf[...] + b2_ref[0]
        yh = y.astype(jnp.bfloat16)
        yl = (y - yh.astype(jnp.float32)).astype(jnp.bfloat16)
        P = PS[...]
        ycat = jnp.concatenate([yh, yl], axis=0)
        Pcat = jnp.concatenate([P, P], axis=0)
        contrib = jax.lax.dot_general(Pcat, ycat, _TN,
                                      preferred_element_type=jnp.float32)

        @pl.when(e == 0)
        def _():
            acc[...] = contrib

        @pl.when(e != 0)
        def _():
            acc[...] = acc[...] + contrib

        @pl.when(e == E - 1)
        def _finish():
            ffn = acc[...] * gcol[...]
            v = jax.lax.dot(ffn.astype(jnp.bfloat16), Wv_ref[...],
                            preferred_element_type=jnp.float32) + bv_ref[0, 0]
            vcol = jnp.clip(v, -V_MAX, V_MAX)

            trow = jax.lax.broadcasted_iota(jnp.int32, (T, _NCH), 0)
            irow = jax.lax.broadcasted_iota(jnp.int32, (T, _NCH), 1)
            selc = (trow // _L == irow).astype(jnp.bfloat16)
            tl = jax.lax.broadcasted_iota(jnp.int32, (T, _L), 0)
            jl = jax.lax.broadcasted_iota(jnp.int32, (T, _L), 1)
            lmb = (tl % _L == jl).astype(jnp.bfloat16)
            lmaskf = (tl % _L == jl).astype(jnp.float32)

            vh = vcol.astype(jnp.bfloat16)
            vl = (vcol - vh.astype(jnp.float32)).astype(jnp.bfloat16)
            vm = jnp.concatenate([vh * lmb, vl * lmb], axis=1)
            v32 = jax.lax.dot_general(selc, vm, _TN,
                                      preferred_element_type=jnp.float32)
            v16 = v32[:, :_L] + v32[:, _L:]

            mrow = maskT_ref[...]
            icol = jax.lax.broadcasted_iota(jnp.int32, (_NCH, T), 0)
            tcol = jax.lax.broadcasted_iota(jnp.int32, (_NCH, T), 1)
            selm = ((tcol // _L == icol).astype(jnp.bfloat16)
                    * mrow.astype(jnp.bfloat16))
            m16 = jax.lax.dot(selm, lmb,
                              preferred_element_type=jnp.float32)

            Ef = _hs_exclusive(_chunk_transforms(v16, _FWD), down=False)
            Eb = _hs_exclusive(_chunk_transforms(v16, _BWD), down=True)
            c0fr, c0fi = _carry0(Ef)
            c0br, c0bi = _carry0(Eb)
            Lr, Li = _replay(v16, c0fr, c0fi, _FWD)
            Rr, Ri = _replay(v16, c0br, c0bi, _BWD)
            dr = -v16 - Lr - Rr
            di = 1.0 - Li - Ri
            nn = dr * dr + di * di
            fr = jnp.clip((dr / nn) * m16,
                          -FEAT_CLAMP, FEAT_CLAMP).astype(jnp.bfloat16)
            fi = jnp.clip((-di / nn) * m16,
                          -FEAT_CLAMP, FEAT_CLAMP).astype(jnp.bfloat16)

            fcat = jnp.concatenate([fr, fi], axis=1)
            u = jax.lax.dot(selc, fcat,
                            preferred_element_type=jnp.float32)
            frb = jnp.sum(u[:, :_L] * lmaskf, axis=1, keepdims=True)
            fib = jnp.sum(u[:, _L:] * lmaskf, axis=1, keepdims=True)

            wo = Wo_ref[...].astype(jnp.bfloat16).astype(jnp.float32)
            spec = frb * wo[0:1, :] + fib * wo[1:2, :] + bo_ref[...]
            out_ref[...] = ffn + bk_ref[0, 0] * spec


def _normalize(t):
    Ar, Ai, Br, Bi, Cr, Ci, Dr, Di = t
    n2 = jnp.maximum(jnp.maximum(Ar * Ar + Ai * Ai, Br * Br + Bi * Bi),
                     jnp.maximum(Cr * Cr + Ci * Ci, Dr * Dr + Di * Di))
    s = jax.lax.rsqrt(n2)
    return tuple(x * s for x in t)


def _chunk_transforms(vv, order):
    one = jnp.ones((_NCH, 1), jnp.float32)
    zero = jnp.zeros((_NCH, 1), jnp.float32)
    Ar, Ai, Br, Bi, Cr, Ci, Dr, Di = one, zero, zero, zero, zero, zero, one, zero
    for j in order:
        wr = -vv[:, j:j + 1]
        nCr = wr * Cr - Ci - Ar
        nCi = wr * Ci + Cr - Ai
        nDr = wr * Dr - Di - Br
        nDi = wr * Di + Dr - Bi
        Ar, Ai, Br, Bi, Cr, Ci, Dr, Di = Cr, Ci, Dr, Di, nCr, nCi, nDr, nDi
    return _normalize((Ar, Ai, Br, Bi, Cr, Ci, Dr, Di))


def _compose(X, Y):
    XAr, XAi, XBr, XBi, XCr, XCi, XDr, XDi = X
    YAr, YAi, YBr, YBi, YCr, YCi, YDr, YDi = Y

    def cmul(ar, ai, br, bi):
        return ar * br - ai * bi, ar * bi + ai * br

    def centry(ar, ai, br, bi, cr, ci, dr, di):
        p, q = cmul(ar, ai, cr, ci)
        r_, s_ = cmul(br, bi, dr, di)
        return p + r_, q + s_

    nAr, nAi = centry(XAr, XAi, XBr, XBi, YAr, YAi, YCr, YCi)
    nBr, nBi = centry(XAr, XAi, XBr, XBi, YBr, YBi, YDr, YDi)
    nCr, nCi = centry(XCr, XCi, XDr, XDi, YAr, YAi, YCr, YCi)
    nDr, nDi = centry(XCr, XCi, XDr, XDi, YBr, YBi, YDr, YDi)
    return (nAr, nAi, nBr, nBi, nCr, nCi, nDr, nDi)


_IDENT = (1.0, 0.0, 0.0, 0.0, 0.0, 0.0, 1.0, 0.0)


def _hs_exclusive(Tm, down):
    rowi = jax.lax.broadcasted_iota(jnp.int32, (_NCH, 1), 0)

    def shifted(X, s):
        if down:
            return tuple(jnp.where(rowi < _NCH - s, jnp.roll(x, -s, axis=0),
                                   idv) for x, idv in zip(X, _IDENT))
        return tuple(jnp.where(rowi >= s, jnp.roll(x, s, axis=0), idv)
                     for x, idv in zip(X, _IDENT))

    X = Tm
    s = 1
    while s < _NCH:
        X = _normalize(_compose(X, shifted(X, s)))
        s *= 2
    return shifted(X, 1)


def _carry0(Ex):
    (_, _, Br, Bi, _, _, Dr, Di) = Ex
    n = Dr * Dr + Di * Di
    return (Br * Dr + Bi * Di) / n, (Bi * Dr - Br * Di) / n


def _replay(vv, c0r, c0i, order):
    cr, ci = c0r, c0i
    cols_r, cols_i = [None] * _L, [None] * _L
    for j in order:
        cols_r[j] = cr
        cols_i[j] = ci
        dr = -vv[:, j:j + 1] - cr
        di = 1.0 - ci
        n = dr * dr + di * di
        cr = dr / n
        ci = -di / n
    return (jnp.concatenate(cols_r, axis=1),
            jnp.concatenate(cols_i, axis=1))


_FWD = tuple(range(_L))
_BWD = tuple(reversed(range(_L)))


def _scanepi_kernel(v16_ref, ffn_ref, mk_ref, Wo_ref, bo_ref, bk_ref,
                    out_ref):
    v = v16_ref[...]
    Ef = _hs_exclusive(_chunk_transforms(v, _FWD), down=False)
    Eb = _hs_exclusive(_chunk_transforms(v, _BWD), down=True)
    c0fr, c0fi = _carry0(Ef)
    c0br, c0bi = _carry0(Eb)
    Lr, Li = _replay(v, c0fr, c0fi, _FWD)
    Rr, Ri = _replay(v, c0br, c0bi, _BWD)

    dr = -v - Lr - Rr
    di = 1.0 - Li - Ri
    n = dr * dr + di * di
    m = mk_ref[...]
    fr = jnp.clip((dr / n) * m, -FEAT_CLAMP, FEAT_CLAMP).astype(jnp.bfloat16)
    fi = jnp.clip((-di / n) * m, -FEAT_CLAMP, FEAT_CLAMP).astype(jnp.bfloat16)

    trow = jax.lax.broadcasted_iota(jnp.int32, (T, _NCH), 0)
    irow = jax.lax.broadcasted_iota(jnp.int32, (T, _NCH), 1)
    sel = (trow // _L == irow).astype(jnp.bfloat16)
    tl = jax.lax.broadcasted_iota(jnp.int32, (T, _L), 0)
    jl = jax.lax.broadcasted_iota(jnp.int32, (T, _L), 1)
    lmask = (tl % _L == jl).astype(jnp.float32)

    def to_col(f16):
        u = jax.lax.dot(sel, f16, preferred_element_type=jnp.float32)
        return jnp.sum(u * lmask, axis=1, keepdims=True)

    frb = to_col(fr)
    fib = to_col(fi)
    wo = Wo_ref[...].astype(jnp.bfloat16).astype(jnp.float32)
    spec = frb * wo[0:1, :] + fib * wo[1:2, :] + bo_ref[...]
    out_ref[...] = ffn_ref[...] + bk_ref[0, 0] * spec


def kernel(x, Wr, br, W1, b1, W2, b2, Wv, bv, Wg, bg, Wo, bo, bk_scale,
           interpret=False):
    f32 = jnp.float32
    x2d = x.reshape(T, D)
    Wcat = jnp.concatenate([Wr, Wg, jnp.zeros((D, 7), f32)], axis=1)
    WcatT = Wcat.T.astype(jnp.bfloat16)
    bcat = jnp.concatenate([br, bg, jnp.zeros((7,), f32)]).reshape(16, 1)

    out2d, maskT, sp, nc = pl.pallas_call(
        _moe_kernel,
        grid=(E, _NF),
        in_specs=[
            pl.BlockSpec((T, D), lambda e, f: (0, 0)),
            pl.BlockSpec((16, D), lambda e, f: (0, 0)),
            pl.BlockSpec((16, 1), lambda e, f: (0, 0)),
            pl.BlockSpec((1, D, _FB), lambda e, f: (e, 0, f)),
            pl.BlockSpec((1, 1, _FB), lambda e, f: (e, 0, f)),
            pl.BlockSpec((1, _FB, D), lambda e, f: (e, f, 0)),
            pl.BlockSpec((1, 1, D), lambda e, f: (e, 0, 0)),
            pl.BlockSpec((D, 1), lambda e, f: (0, 0)),
            pl.BlockSpec((1, 1), lambda e, f: (0, 0)),
            pl.BlockSpec((2, D), lambda e, f: (0, 0)),
            pl.BlockSpec((1, D), lambda e, f: (0, 0)),
            pl.BlockSpec((1, 1), lambda e, f: (0, 0)),
        ],
        out_specs=[
            pl.BlockSpec((T, D), lambda e, f: (0, 0)),
            pl.BlockSpec((1, T), lambda e, f: (0, 0)),
            pl.BlockSpec((1, 1), lambda e, f: (0, 0)),
            pl.BlockSpec((1, 1), lambda e, f: (0, 0)),
        ],
        out_shape=[
            jax.ShapeDtypeStruct((T, D), f32),
            jax.ShapeDtypeStruct((1, T), f32),
            jax.ShapeDtypeStruct((1, 1), f32),
            jax.ShapeDtypeStruct((1, 1), f32),
        ],
        scratch_shapes=[
            pltpu.VMEM((T, D), jnp.bfloat16),
            pltpu.VMEM((1, T), jnp.int32),
            pltpu.VMEM((1, T), jnp.int32),
            pltpu.VMEM((T, 1), f32),
            pltpu.VMEM((C, T), jnp.bfloat16),
            pltpu.VMEM((C, D), jnp.bfloat16),
            pltpu.VMEM((C, D), f32),
            pltpu.VMEM((T, D), f32),
        ],
        compiler_params=pltpu.CompilerParams(
            dimension_semantics=("arbitrary", "arbitrary")),
        interpret=interpret,
    )(x2d, WcatT, bcat, W1, b1.reshape(E, 1, F), W2, b2.reshape(E, 1, D),
      Wv.astype(jnp.bfloat16), bv.reshape(1, 1), Wo, bo.reshape(1, D),
      bk_scale.reshape(1, 1))

    output = out2d.reshape(1, T, D)
    return output, maskT, sp.reshape(()), nc.reshape(())


# speedup vs baseline: 1.0822x; 1.0822x over previous
import jax
import jax.numpy as jnp
import numpy as np
from jax.experimental import pallas as pl
from jax.experimental.pallas import tpu as pltpu

T = 2048
D = 768
E = 8
F = 4 * D
C = int(np.ceil(T / E * 1.25))
KKEEP = max(1, int(round(T * (1.0 - 0.6))))
V_MAX = 3.0
FEAT_CLAMP = 10.0
_L = 16
_NCH = T // _L
_NF = 2
_FB = F // _NF

_NT = (((1,), (1,)), ((), ()))
_TN = (((0,), (0,)), ((), ()))


def _moe_kernel(x_ref, WcatT_ref, bcat_ref, W1_ref, b1_ref, W2_ref, b2_ref,
                Wv_ref, bv_ref, Wo_ref, bo_ref, bk_ref,
                out_ref, maskT_ref, sp_ref, nc_ref,
                xbf, eS, pS, gcol, PS, bufS, ybuf):
    e = pl.program_id(0)
    f = pl.program_id(1)

    @pl.when((e == 0) & (f == 0))
    def _route():
        xb = x_ref[...].astype(jnp.bfloat16)
        xbf[...] = xb
        wct = WcatT_ref[...]
        lgT = jax.lax.dot_general(
            wct, xb, _NT, preferred_element_type=jnp.float32) + bcat_ref[...]
        lg = lgT[:E, :]
        mx = jnp.max(lg, axis=0, keepdims=True)
        subi = jax.lax.broadcasted_iota(jnp.int32, (E, T), 0)
        eT = jnp.min(jnp.where(lg == mx, subi, E), axis=0, keepdims=True)
        eS[...] = eT

        onehot = (subi == eT).astype(jnp.float32)
        lane = jax.lax.broadcasted_iota(jnp.int32, (E, T), 1)
        cs = onehot
        sft = 1
        while sft < T:
            r = jnp.roll(cs, sft, axis=1)
            cs = cs + jnp.where(lane >= sft, r, 0.0)
            sft *= 2
        posT = jnp.sum(cs * onehot, axis=0, keepdims=True) - 1.0
        pS[...] = posT.astype(jnp.int32)

        lgC = jax.lax.dot_general(
            xb, wct, _NT, preferred_element_type=jnp.float32) \
            + bcat_ref[...].reshape(1, 16)
        r8 = lgC[:, :E]
        mxc = jnp.max(r8, axis=1, keepdims=True)
        sc_ = jnp.sum(jnp.exp(r8 - mxc), axis=1, keepdims=True)
        gcol[...] = 1.0 / sc_

        sc = lgT[E:E + 1, :]
        bi = jax.lax.bitcast_convert_type(sc, jnp.int32)
        key = jnp.where(bi < 0, bi ^ jnp.int32(0x7FFFFFFF), bi)

        def count_ge(m):
            return jnp.sum((key >= m).astype(jnp.float32))

        kf = jnp.float32(KKEEP)
        m0 = jnp.where(count_ge(jnp.int32(0)) >= kf,
                       jnp.int32(0), jnp.int32(-2147483648))

        def body(i, m):
            mt = m | jax.lax.shift_left(jnp.int32(1), 30 - i)
            return jnp.where(count_ge(mt) >= kf, mt, m)

        m = jax.lax.fori_loop(0, 31, body, m0)
        maskT = (key >= m).astype(jnp.float32)
        maskT_ref[...] = maskT
        nc = jnp.sum(maskT, axis=1, keepdims=True)
        nc_ref[...] = nc
        sp_ref[...] = 1.0 - nc / jnp.float32(T)

    @pl.when(f == 0)
    def _dispatch():
        iota0 = jax.lax.broadcasted_iota(jnp.int32, (C, T), 0)
        pm = jnp.where(eS[...] == e, pS[...], -1)
        P = (iota0 == pm).astype(jnp.bfloat16)
        PS[...] = P
        bufS[...] = jax.lax.dot(
            P, xbf[...], preferred_element_type=jnp.float32
        ).astype(jnp.bfloat16)

    w1 = W1_ref[0].astype(jnp.bfloat16)
    h = jax.nn.relu(jax.lax.dot(bufS[...], w1,
                                preferred_element_type=jnp.float32)
                    + b1_ref[0])
    w2 = W2_ref[0].astype(jnp.bfloat16)
    part = jax.lax.dot(h.astype(jnp.bfloat16), w2,
                       preferred_element_type=jnp.float32)

    @pl.when(f == 0)
    def _():
        ybuf[...] = part

    @pl.when(f != 0)
    def _():
        ybuf[...] = ybuf[...] + part

    @pl.when(f == _NF - 1)
    def _combine():
        y = ybuf[...] + b2_ref[0]
        yh = y.astype(jnp.bfloat16)
        yl = (y - yh.astype(jnp.float32)).astype(jnp.bfloat16)
        P = PS[...]
        ycat = jnp.concatenate([yh, yl], axis=0)
        Pcat = jnp.concatenate([P, P], axis=0)
        contrib = jax.lax.dot_general(Pcat, ycat, _TN,
                                      preferred_element_type=jnp.float32)

        @pl.when(e == 0)
        def _():
            out_ref[...] = contrib

        @pl.when(e != 0)
        def _():
            out_ref[...] = out_ref[...] + contrib

        @pl.when(e == E - 1)
        def _finish():
            ffn = out_ref[...] * gcol[...]
            v = jax.lax.dot(ffn.astype(jnp.bfloat16), Wv_ref[...],
                            preferred_element_type=jnp.float32) + bv_ref[0, 0]
            vcol = jnp.clip(v, -V_MAX, V_MAX)

            trow = jax.lax.broadcasted_iota(jnp.int32, (T, _NCH), 0)
            irow = jax.lax.broadcasted_iota(jnp.int32, (T, _NCH), 1)
            selc = (trow // _L == irow).astype(jnp.bfloat16)
            tl = jax.lax.broadcasted_iota(jnp.int32, (T, _L), 0)
            jl = jax.lax.broadcasted_iota(jnp.int32, (T, _L), 1)
            lmb = (tl % _L == jl).astype(jnp.bfloat16)
            lmaskf = (tl % _L == jl).astype(jnp.float32)

            vh = vcol.astype(jnp.bfloat16)
            vl = (vcol - vh.astype(jnp.float32)).astype(jnp.bfloat16)
            vm = jnp.concatenate([vh * lmb, vl * lmb], axis=1)
            v32 = jax.lax.dot_general(selc, vm, _TN,
                                      preferred_element_type=jnp.float32)
            v16 = v32[:, :_L] + v32[:, _L:]

            mrow = maskT_ref[...]
            icol = jax.lax.broadcasted_iota(jnp.int32, (_NCH, T), 0)
            tcol = jax.lax.broadcasted_iota(jnp.int32, (_NCH, T), 1)
            selm = ((tcol // _L == icol).astype(jnp.bfloat16)
                    * mrow.astype(jnp.bfloat16))
            m16 = jax.lax.dot(selm, lmb,
                              preferred_element_type=jnp.float32)

            Ef = _hs_exclusive(_chunk_transforms(v16, _FWD), down=False)
            Eb = _hs_exclusive(_chunk_transforms(v16, _BWD), down=True)
            c0fr, c0fi = _carry0(Ef)
            c0br, c0bi = _carry0(Eb)
            Lr, Li = _replay(v16, c0fr, c0fi, _FWD)
            Rr, Ri = _replay(v16, c0br, c0bi, _BWD)
            dr = -v16 - Lr - Rr
            di = 1.0 - Li - Ri
            nn = dr * dr + di * di
            fr = jnp.clip((dr / nn) * m16,
                          -FEAT_CLAMP, FEAT_CLAMP).astype(jnp.bfloat16)
            fi = jnp.clip((-di / nn) * m16,
                          -FEAT_CLAMP, FEAT_CLAMP).astype(jnp.bfloat16)

            fcat = jnp.concatenate([fr, fi], axis=1)
            u = jax.lax.dot(selc, fcat,
                            preferred_element_type=jnp.float32)
            frb = jnp.sum(u[:, :_L] * lmaskf, axis=1, keepdims=True)
            fib = jnp.sum(u[:, _L:] * lmaskf, axis=1, keepdims=True)

            wo = Wo_ref[...].astype(jnp.bfloat16).astype(jnp.float32)
            spec = frb * wo[0:1, :] + fib * wo[1:2, :] + bo_ref[...]
            out_ref[...] = ffn + bk_ref[0, 0] * spec


def _normalize(t):
    Ar, Ai, Br, Bi, Cr, Ci, Dr, Di = t
    n2 = jnp.maximum(jnp.maximum(Ar * Ar + Ai * Ai, Br * Br + Bi * Bi),
                     jnp.maximum(Cr * Cr + Ci * Ci, Dr * Dr + Di * Di))
    s = jax.lax.rsqrt(n2)
    return tuple(x * s for x in t)


def _chunk_transforms(vv, order):
    one = jnp.ones((_NCH, 1), jnp.float32)
    zero = jnp.zeros((_NCH, 1), jnp.float32)
    Ar, Ai, Br, Bi, Cr, Ci, Dr, Di = one, zero, zero, zero, zero, zero, one, zero
    for j in order:
        wr = -vv[:, j:j + 1]
        nCr = wr * Cr - Ci - Ar
        nCi = wr * Ci + Cr - Ai
        nDr = wr * Dr - Di - Br
        nDi = wr * Di + Dr - Bi
        Ar, Ai, Br, Bi, Cr, Ci, Dr, Di = Cr, Ci, Dr, Di, nCr, nCi, nDr, nDi
    return _normalize((Ar, Ai, Br, Bi, Cr, Ci, Dr, Di))


def _compose(X, Y):
    XAr, XAi, XBr, XBi, XCr, XCi, XDr, XDi = X
    YAr, YAi, YBr, YBi, YCr, YCi, YDr, YDi = Y

    def cmul(ar, ai, br, bi):
        return ar * br - ai * bi, ar * bi + ai * br

    def centry(ar, ai, br, bi, cr, ci, dr, di):
        p, q = cmul(ar, ai, cr, ci)
        r_, s_ = cmul(br, bi, dr, di)
        return p + r_, q + s_

    nAr, nAi = centry(XAr, XAi, XBr, XBi, YAr, YAi, YCr, YCi)
    nBr, nBi = centry(XAr, XAi, XBr, XBi, YBr, YBi, YDr, YDi)
    nCr, nCi = centry(XCr, XCi, XDr, XDi, YAr, YAi, YCr, YCi)
    nDr, nDi = centry(XCr, XCi, XDr, XDi, YBr, YBi, YDr, YDi)
    return (nAr, nAi, nBr, nBi, nCr, nCi, nDr, nDi)


_IDENT = (1.0, 0.0, 0.0, 0.0, 0.0, 0.0, 1.0, 0.0)


def _hs_exclusive(Tm, down):
    rowi = jax.lax.broadcasted_iota(jnp.int32, (_NCH, 1), 0)

    def shifted(X, s):
        if down:
            return tuple(jnp.where(rowi < _NCH - s, jnp.roll(x, -s, axis=0),
                                   idv) for x, idv in zip(X, _IDENT))
        return tuple(jnp.where(rowi >= s, jnp.roll(x, s, axis=0), idv)
                     for x, idv in zip(X, _IDENT))

    X = Tm
    s = 1
    while s < _NCH:
        X = _normalize(_compose(X, shifted(X, s)))
        s *= 2
    return shifted(X, 1)


def _carry0(Ex):
    (_, _, Br, Bi, _, _, Dr, Di) = Ex
    n = Dr * Dr + Di * Di
    return (Br * Dr + Bi * Di) / n, (Bi * Dr - Br * Di) / n


def _replay(vv, c0r, c0i, order):
    cr, ci = c0r, c0i
    cols_r, cols_i = [None] * _L, [None] * _L
    for j in order:
        cols_r[j] = cr
        cols_i[j] = ci
        dr = -vv[:, j:j + 1] - cr
        di = 1.0 - ci
        n = dr * dr + di * di
        cr = dr / n
        ci = -di / n
    return (jnp.concatenate(cols_r, axis=1),
            jnp.concatenate(cols_i, axis=1))


_FWD = tuple(range(_L))
_BWD = tuple(reversed(range(_L)))


def _scanepi_kernel(v16_ref, ffn_ref, mk_ref, Wo_ref, bo_ref, bk_ref,
                    out_ref):
    v = v16_ref[...]
    Ef = _hs_exclusive(_chunk_transforms(v, _FWD), down=False)
    Eb = _hs_exclusive(_chunk_transforms(v, _BWD), down=True)
    c0fr, c0fi = _carry0(Ef)
    c0br, c0bi = _carry0(Eb)
    Lr, Li = _replay(v, c0fr, c0fi, _FWD)
    Rr, Ri = _replay(v, c0br, c0bi, _BWD)

    dr = -v - Lr - Rr
    di = 1.0 - Li - Ri
    n = dr * dr + di * di
    m = mk_ref[...]
    fr = jnp.clip((dr / n) * m, -FEAT_CLAMP, FEAT_CLAMP).astype(jnp.bfloat16)
    fi = jnp.clip((-di / n) * m, -FEAT_CLAMP, FEAT_CLAMP).astype(jnp.bfloat16)

    trow = jax.lax.broadcasted_iota(jnp.int32, (T, _NCH), 0)
    irow = jax.lax.broadcasted_iota(jnp.int32, (T, _NCH), 1)
    sel = (trow // _L == irow).astype(jnp.bfloat16)
    tl = jax.lax.broadcasted_iota(jnp.int32, (T, _L), 0)
    jl = jax.lax.broadcasted_iota(jnp.int32, (T, _L), 1)
    lmask = (tl % _L == jl).astype(jnp.float32)

    def to_col(f16):
        u = jax.lax.dot(sel, f16, preferred_element_type=jnp.float32)
        return jnp.sum(u * lmask, axis=1, keepdims=True)

    frb = to_col(fr)
    fib = to_col(fi)
    wo = Wo_ref[...].astype(jnp.bfloat16).astype(jnp.float32)
    spec = frb * wo[0:1, :] + fib * wo[1:2, :] + bo_ref[...]
    out_ref[...] = ffn_ref[...] + bk_ref[0, 0] * spec


def kernel(x, Wr, br, W1, b1, W2, b2, Wv, bv, Wg, bg, Wo, bo, bk_scale,
           interpret=False):
    f32 = jnp.float32
    x2d = x.reshape(T, D)
    Wcat = jnp.concatenate([Wr, Wg, jnp.zeros((D, 7), f32)], axis=1)
    WcatT = Wcat.T.astype(jnp.bfloat16)
    bcat = jnp.concatenate([br, bg, jnp.zeros((7,), f32)]).reshape(16, 1)

    out2d, maskT, sp, nc = pl.pallas_call(
        _moe_kernel,
        grid=(E, _NF),
        in_specs=[
            pl.BlockSpec((T, D), lambda e, f: (0, 0)),
            pl.BlockSpec((16, D), lambda e, f: (0, 0)),
            pl.BlockSpec((16, 1), lambda e, f: (0, 0)),
            pl.BlockSpec((1, D, _FB), lambda e, f: (e, 0, f)),
            pl.BlockSpec((1, 1, _FB), lambda e, f: (e, 0, f)),
            pl.BlockSpec((1, _FB, D), lambda e, f: (e, f, 0)),
            pl.BlockSpec((1, 1, D), lambda e, f: (e, 0, 0)),
            pl.BlockSpec((D, 1), lambda e, f: (0, 0)),
            pl.BlockSpec((1, 1), lambda e, f: (0, 0)),
            pl.BlockSpec((2, D), lambda e, f: (0, 0)),
            pl.BlockSpec((1, D), lambda e, f: (0, 0)),
            pl.BlockSpec((1, 1), lambda e, f: (0, 0)),
        ],
        out_specs=[
            pl.BlockSpec((T, D), lambda e, f: (0, 0)),
            pl.BlockSpec((1, T), lambda e, f: (0, 0)),
            pl.BlockSpec((1, 1), lambda e, f: (0, 0)),
            pl.BlockSpec((1, 1), lambda e, f: (0, 0)),
        ],
        out_shape=[
            jax.ShapeDtypeStruct((T, D), f32),
            jax.ShapeDtypeStruct((1, T), f32),
            jax.ShapeDtypeStruct((1, 1), f32),
            jax.ShapeDtypeStruct((1, 1), f32),
        ],
        scratch_shapes=[
            pltpu.VMEM((T, D), jnp.bfloat16),
            pltpu.VMEM((1, T), jnp.int32),
            pltpu.VMEM((1, T), jnp.int32),
            pltpu.VMEM((T, 1), f32),
            pltpu.VMEM((C, T), jnp.bfloat16),
            pltpu.VMEM((C, D), jnp.bfloat16),
            pltpu.VMEM((C, D), f32),
        ],
        compiler_params=pltpu.CompilerParams(
            dimension_semantics=("arbitrary", "arbitrary")),
        interpret=interpret,
    )(x2d, WcatT, bcat, W1, b1.reshape(E, 1, F), W2, b2.reshape(E, 1, D),
      Wv.astype(jnp.bfloat16), bv.reshape(1, 1), Wo, bo.reshape(1, D),
      bk_scale.reshape(1, 1))

    output = out2d.reshape(1, T, D)
    return output, maskT, sp.reshape(()), nc.reshape(())
